# Initial kernel scaffold; baseline (speedup 1.0000x reference)
import jax
import jax.numpy as jnp
from jax.experimental import pallas as pl


def kernel(x, y, zetas):
    raise NotImplementedError("write your pallas kernel here")


# speedup vs baseline: 1.3887x; 1.3887x over previous
import functools

import jax
import jax.numpy as jnp
from jax import lax
from jax.experimental import pallas as pl
from jax.experimental.pallas import tpu as pltpu
from jax.experimental.pallas import tpu_sc as plsc

_NC = 2
_NS = 16
_NW = _NC * _NS
_LANES = 16
_CHUNK = 128


@functools.cache
def _build(batch: int):
    P = batch // _NW
    NCH = P // _CHUNK
    NV = P // _LANES
    CPR = _CHUNK // _LANES

    mesh = plsc.VectorSubcoreMesh(
        core_axis_name="c", subcore_axis_name="s",
        num_cores=_NC, num_subcores=_NS)

    @functools.partial(
        pl.kernel,
        out_type=jax.ShapeDtypeStruct((batch,), jnp.float32),
        mesh=mesh,
        scratch_types=[
            pltpu.VMEM((P,), jnp.int32),
            pltpu.VMEM((P,), jnp.int32),
            pltpu.VMEM((NCH, _CHUNK), jnp.int32),
            pltpu.VMEM((NCH, _CHUNK), jnp.int32),
            pltpu.VMEM((NCH, _CHUNK), jnp.float32),
            pltpu.VMEM((NCH, _CHUNK), jnp.float32),
            pltpu.VMEM((P,), jnp.float32),
            pltpu.VMEM((P,), jnp.float32),
            pltpu.VMEM((P,), jnp.float32),
            pltpu.VMEM((P,), jnp.float32),
            pltpu.SemaphoreType.DMA,
        ],
    )
    def body(ia_hbm, ib_hbm, y_hbm, zetas_hbm, out_hbm,
             av, bv, ga, gb, za, zb, ma, mb, yv, lv, sem):
        wid = lax.axis_index("s") * _NC + lax.axis_index("c")
        base = wid * P
        pltpu.sync_copy(ia_hbm.at[pl.ds(base, P)], av)
        pltpu.sync_copy(ib_hbm.at[pl.ds(base, P)], bv)
        pltpu.sync_copy(y_hbm.at[pl.ds(base, P)], yv)

        for i in range(NV):
            sl = pl.ds(i * _LANES, _LANES)
            ca = av[sl]
            cb = bv[sl]
            r, c = divmod(i, CPR)
            csl = pl.ds(c * _LANES, _LANES)
            ga[r, csl] = jnp.maximum(ca - 1, 0)
            gb[r, csl] = jnp.maximum(cb - 1, 0)
            ma[sl] = jnp.where(ca == 0, 0.0, 1.0).astype(jnp.float32)
            mb[sl] = jnp.where(cb == 0, 0.0, 1.0).astype(jnp.float32)

        copies = []
        for r in range(NCH):
            copies.append(pltpu.async_copy(zetas_hbm.at[ga.at[r]], za.at[r], sem))
            copies.append(pltpu.async_copy(zetas_hbm.at[gb.at[r]], zb.at[r], sem))
        for cp in copies:
            cp.wait()

        for i in range(NV):
            r, c = divmod(i, CPR)
            csl = pl.ds(c * _LANES, _LANES)
            sl = pl.ds(i * _LANES, _LANES)
            zia = za[r, csl] * ma[sl]
            zib = zb[r, csl] * mb[sl]
            logit = zia - zib
            m = jnp.maximum(logit, 0.0)
            t = jnp.exp(-jnp.abs(logit))
            s = t / (t + 2.0)
            s2 = s * s
            log1p_t = 2.0 * s * (1.0 + s2 * (
                (1.0 / 3.0) + s2 * (0.2 + s2 * ((1.0 / 7.0) + s2 * (1.0 / 9.0)))))
            lv[sl] = m + log1p_t - yv[sl] * logit

        pltpu.sync_copy(lv, out_hbm.at[pl.ds(base, P)])

    return body


def kernel(x, y, zetas):
    batch = x.shape[0]
    xi = x.astype(jnp.int32)
    return _build(batch)(xi[:, 0], xi[:, 1],
                         y.astype(jnp.float32), zetas.astype(jnp.float32))


# speedup vs baseline: 1.4638x; 1.0540x over previous
import functools

import jax
import jax.numpy as jnp
from jax import lax
from jax.experimental import pallas as pl
from jax.experimental.pallas import tpu as pltpu
from jax.experimental.pallas import tpu_sc as plsc

_NC = 2
_NS = 16
_NW = _NC * _NS
_LANES = 16
_CHUNK = 128


@functools.cache
def _build(batch: int):
    P = batch // _NW
    NCH = P // _CHUNK
    CPR = _CHUNK // _LANES

    mesh = plsc.VectorSubcoreMesh(
        core_axis_name="c", subcore_axis_name="s",
        num_cores=_NC, num_subcores=_NS)

    @functools.partial(
        pl.kernel,
        out_type=jax.ShapeDtypeStruct((batch,), jnp.float32),
        mesh=mesh,
        scratch_types=[
            pltpu.VMEM((P,), jnp.int32),
            pltpu.VMEM((P,), jnp.int32),
            pltpu.VMEM((NCH, _CHUNK), jnp.int32),
            pltpu.VMEM((NCH, _CHUNK), jnp.int32),
            pltpu.VMEM((NCH, _CHUNK), jnp.float32),
            pltpu.VMEM((NCH, _CHUNK), jnp.float32),
            pltpu.VMEM((P,), jnp.float32),
            pltpu.VMEM((P,), jnp.float32),
            pltpu.SemaphoreType.DMA,
            pltpu.SemaphoreType.DMA((NCH,)),
            pltpu.SemaphoreType.DMA,
        ],
    )
    def body(ia_hbm, ib_hbm, y_hbm, zetas_hbm, out_hbm,
             av, bv, ga, gb, za, zb, yv, lv, isem, gsem, osem):
        wid = lax.axis_index("s") * _NC + lax.axis_index("c")
        base = wid * P
        in_a = pltpu.async_copy(ia_hbm.at[pl.ds(base, P)], av, isem)
        in_b = pltpu.async_copy(ib_hbm.at[pl.ds(base, P)], bv, isem)
        in_y = pltpu.async_copy(y_hbm.at[pl.ds(base, P)], yv, isem)
        in_a.wait()
        in_b.wait()

        gathers = []
        for r in range(NCH):
            for k in range(CPR):
                sl = pl.ds((r * CPR + k) * _LANES, _LANES)
                csl = pl.ds(k * _LANES, _LANES)
                ga[r, csl] = jnp.maximum(av[sl] - 1, 0)
                gb[r, csl] = jnp.maximum(bv[sl] - 1, 0)
            gathers.append((
                pltpu.async_copy(zetas_hbm.at[ga.at[r]], za.at[r], gsem.at[r]),
                pltpu.async_copy(zetas_hbm.at[gb.at[r]], zb.at[r], gsem.at[r]),
            ))
        in_y.wait()

        out_copies = []
        for r in range(NCH):
            cpa, cpb = gathers[r]
            cpa.wait()
            cpb.wait()
            for k in range(CPR):
                i = r * CPR + k
                sl = pl.ds(i * _LANES, _LANES)
                csl = pl.ds(k * _LANES, _LANES)
                zia = jnp.where(av[sl] == 0, 0.0, za[r, csl])
                zib = jnp.where(bv[sl] == 0, 0.0, zb[r, csl])
                logit = zia - zib
                m = jnp.maximum(logit, 0.0)
                t = jnp.exp(-jnp.abs(logit))
                s = t / (t + 2.0)
                s2 = s * s
                log1p_t = 2.0 * s * (1.0 + s2 * (
                    (1.0 / 3.0) + s2 * (0.2 + s2 * (
                        (1.0 / 7.0) + s2 * (1.0 / 9.0)))))
                lv[sl] = m + log1p_t - yv[sl] * logit
            out_copies.append(pltpu.async_copy(
                lv.at[pl.ds(r * _CHUNK, _CHUNK)],
                out_hbm.at[pl.ds(base + r * _CHUNK, _CHUNK)], osem))
        for cp in out_copies:
            cp.wait()

    return body


def kernel(x, y, zetas):
    batch = x.shape[0]
    xi = x.astype(jnp.int32)
    return _build(batch)(xi[:, 0], xi[:, 1],
                         y.astype(jnp.float32), zetas.astype(jnp.float32))
